# baseline (speedup 1.0000x reference)
import functools
import jax
import jax.numpy as jnp
from jax import lax
from jax.experimental import pallas as pl
from jax.experimental.pallas import tpu as pltpu, tpu_sc as plsc

N = 10000
D = 128
F = 64
E = 320000
NC = 2
NS = 16
NW = NC * NS
EPW = E // NW
CH = 128
NFULL = EPW // CH
TAIL = EPW - NFULL * CH
CHP = 80
NCHP = EPW // CHP

_mesh = functools.partial(
    plsc.VectorSubcoreMesh, core_axis_name="c", subcore_axis_name="s")


def _wid():
    return lax.axis_index("c") * NS + lax.axis_index("s")


@functools.partial(
    pl.kernel,
    out_type=jax.ShapeDtypeStruct((NW, N), jnp.float32),
    mesh=_mesh(),
    compiler_params=pltpu.CompilerParams(needs_layout_passes=False, use_tc_tiling_on_sc=False),
    scratch_types=[
        pltpu.VMEM((EPW,), jnp.int32),
        pltpu.VMEM((N,), jnp.float32),
        pltpu.SemaphoreType.DMA,
    ],
)
def _sc_deg(dst_hbm, out_hbm, dstv, degv, sem):
    wid = _wid()
    base = wid * EPW

    zero16 = jnp.zeros((16,), jnp.float32)

    def zbody(i, _):
        degv[pl.ds(i * 16, 16)] = zero16
        return 0
    lax.fori_loop(0, N // 16, zbody, 0)

    pltpu.sync_copy(dst_hbm.at[pl.ds(base, EPW)], dstv)

    one16 = jnp.ones((16,), jnp.float32)

    def body(i, _):
        idx = dstv[pl.ds(i * 16, 16)]
        plsc.addupdate_scatter(degv, [idx], one16)
        return 0
    lax.fori_loop(0, EPW // 16, body, 0)

    pltpu.sync_copy(degv, out_hbm.at[wid])


@functools.partial(
    pl.kernel,
    out_type=jax.ShapeDtypeStruct((NC, N, F), jnp.float32),
    mesh=_mesh(),
    compiler_params=pltpu.CompilerParams(needs_layout_passes=False, use_tc_tiling_on_sc=False),
    scratch_types=[
        pltpu.VMEM((NCHP, CHP), jnp.int32),
        pltpu.VMEM((NCHP, CHP), jnp.int32),
        pltpu.VMEM((CHP, F), jnp.float32),
        pltpu.VMEM((CHP, F), jnp.float32),
        pltpu.VMEM_SHARED((N, F), jnp.float32),
        pltpu.VMEM_SHARED((N, F), jnp.float32),
        pltpu.SemaphoreType.DMA,
        pltpu.SemaphoreType.DMA,
    ],
)
def _sc_prop(tp_hbm, src2_hbm, dst2_hbm, zeros_hbm, out_hbm,
             srcv, dstv, rowsa, rowsb, acc, tsh, sema, semb):
    c = lax.axis_index("c")
    s = lax.axis_index("s")
    row0 = _wid() * NCHP

    @pl.when(s < 10)
    def _():
        pltpu.sync_copy(zeros_hbm.at[pl.ds(s * 1000, 1000)],
                        acc.at[pl.ds(s * 1000, 1000)])
        pltpu.sync_copy(tp_hbm.at[pl.ds(s * 1000, 1000)],
                        tsh.at[pl.ds(s * 1000, 1000)])
    pltpu.sync_copy(src2_hbm.at[pl.ds(row0, NCHP)], srcv)
    pltpu.sync_copy(dst2_hbm.at[pl.ds(row0, NCHP)], dstv)
    plsc.subcore_barrier()

    def start(j, buf, sem):
        pltpu.async_copy(tsh.at[srcv.at[j]], buf, sem)

    def wait(buf, sem):
        pltpu.make_async_copy(tsh.at[srcv.at[0]], buf, sem).wait()

    def scat(j, buf):
        pltpu.sync_copy(buf, acc.at[dstv.at[j]], add=True)

    start(0, rowsa, sema)

    def pair(jj, _):
        j = 2 * jj
        start(j + 1, rowsb, semb)
        wait(rowsa, sema)
        scat(j, rowsa)

        @pl.when(jj < NCHP // 2 - 1)
        def _():
            start(j + 2, rowsa, sema)
        wait(rowsb, semb)
        scat(j + 1, rowsb)
        return 0
    lax.fori_loop(0, NCHP // 2, pair, 0)

    start(NCHP - 1, rowsa, sema)
    wait(rowsa, sema)
    scat(NCHP - 1, rowsa)

    plsc.subcore_barrier()

    @pl.when(s < 10)
    def _():
        pltpu.sync_copy(acc.at[pl.ds(s * 1000, 1000)],
                        out_hbm.at[c, pl.ds(s * 1000, 1000)])


@functools.partial(
    pl.kernel,
    out_type=jax.ShapeDtypeStruct((E,), jnp.float32),
    mesh=_mesh(),
    compiler_params=pltpu.CompilerParams(needs_layout_passes=False, use_tc_tiling_on_sc=False),
    scratch_types=[
        pltpu.VMEM((NCHP, CHP), jnp.int32),
        pltpu.VMEM((NCHP, CHP), jnp.int32),
        pltpu.VMEM((CHP, F), jnp.float32),
        pltpu.VMEM((CHP, F), jnp.float32),
        pltpu.VMEM((CHP, F), jnp.float32),
        pltpu.VMEM((CHP, F), jnp.float32),
        pltpu.VMEM((EPW,), jnp.float32),
        pltpu.VMEM((16,), jnp.float32),
        pltpu.VMEM_SHARED((N, F), jnp.float32),
        pltpu.VMEM_SHARED((N, F), jnp.float32),
        pltpu.SemaphoreType.DMA,
        pltpu.SemaphoreType.DMA,
    ],
)
def _sc_score(u_hbm, h_hbm, src2_hbm, dst2_hbm, bb_hbm, out_hbm,
              srcv, dstv, ua, ha, ub, hb, scores, bbv, ush, hsh, sema, semb):
    s = lax.axis_index("s")
    row0 = _wid() * NCHP
    pltpu.sync_copy(bb_hbm, bbv)
    bbvec = bbv[...]
    pltpu.sync_copy(src2_hbm.at[pl.ds(row0, NCHP)], srcv)
    pltpu.sync_copy(dst2_hbm.at[pl.ds(row0, NCHP)], dstv)

    @pl.when(s < 10)
    def _():
        pltpu.sync_copy(u_hbm.at[pl.ds(s * 1000, 1000)],
                        ush.at[pl.ds(s * 1000, 1000)])
        pltpu.sync_copy(h_hbm.at[pl.ds(s * 1000, 1000)],
                        hsh.at[pl.ds(s * 1000, 1000)])
    plsc.subcore_barrier()

    iota16 = lax.iota(jnp.int32, 16)

    def start(j, ubuf, hbuf, sem):
        pltpu.async_copy(ush.at[srcv.at[j]], ubuf, sem)
        pltpu.async_copy(hsh.at[dstv.at[j]], hbuf, sem)

    def wait(ubuf, hbuf, sem):
        pltpu.make_async_copy(ush.at[srcv.at[0]], ubuf, sem).wait()
        pltpu.make_async_copy(hsh.at[dstv.at[0]], hbuf, sem).wait()

    def compute(j, ubuf, hbuf):
        def group(g, _):
            rowi = g * 16 + iota16
            acc = jnp.zeros((16,), jnp.float32)
            for f in range(F):
                coli = jnp.full((16,), f, jnp.int32)
                acc = acc + (plsc.load_gather(ubuf, [rowi, coli])
                             * plsc.load_gather(hbuf, [rowi, coli]))
            scores[pl.ds(j * CHP + g * 16, 16)] = acc + bbvec
            return 0
        lax.fori_loop(0, CHP // 16, group, 0)

    start(0, ua, ha, sema)

    def pair(jj, _):
        j = 2 * jj
        start(j + 1, ub, hb, semb)
        wait(ua, ha, sema)
        compute(j, ua, ha)

        @pl.when(jj < NCHP // 2 - 1)
        def _():
            start(j + 2, ua, ha, sema)
        wait(ub, hb, semb)
        compute(j + 1, ub, hb)
        return 0
    lax.fori_loop(0, NCHP // 2, pair, 0)

    start(NCHP - 1, ua, ha, sema)
    wait(ua, ha, sema)
    compute(NCHP - 1, ua, ha)

    pltpu.sync_copy(scores, out_hbm.at[pl.ds(_wid() * EPW, EPW)])


def _tc1_body(x_ref, w0_ref, b0_ref, w1_ref, deg_ref, t1p_ref, dinv_ref):
    h0 = jnp.dot(x_ref[...], w0_ref[...],
                 preferred_element_type=jnp.float32) + b0_ref[...]
    t1 = jnp.dot(h0, w1_ref[...], preferred_element_type=jnp.float32)
    deg = jnp.sum(deg_ref[...], axis=1, keepdims=True) + 1.0
    dinv = lax.rsqrt(deg)
    dinv_ref[...] = dinv
    t1p_ref[...] = t1 * dinv


def _tc1(x, W0, b0, W1, deg3):
    return pl.pallas_call(
        _tc1_body,
        out_shape=[jax.ShapeDtypeStruct((N, F), jnp.float32),
                   jax.ShapeDtypeStruct((N, 1), jnp.float32)],
    )(x, W0, b0, W1, deg3)


def _tc2_body(pa_ref, pb_ref, tp_ref, dinv_ref, b1_ref, w2_ref, out_ref):
    dinv = dinv_ref[...]
    ssum = pa_ref[...] + pb_ref[...] + tp_ref[...]
    h1 = jnp.maximum(ssum * dinv + b1_ref[...], 0.0)
    t2 = jnp.dot(h1, w2_ref[...], preferred_element_type=jnp.float32)
    out_ref[...] = t2 * dinv


def _tc2(pa, pb, tp, dinv, b1, W2):
    return pl.pallas_call(
        _tc2_body,
        out_shape=jax.ShapeDtypeStruct((N, F), jnp.float32),
    )(pa, pb, tp, dinv, b1, W2)


def _tc3_body(pa_ref, pb_ref, tp_ref, dinv_ref, b2_ref, wb_ref,
              h2_ref, u_ref):
    ssum = pa_ref[...] + pb_ref[...] + tp_ref[...]
    h2 = ssum * dinv_ref[...] + b2_ref[...]
    h2_ref[...] = h2
    u_ref[...] = jnp.dot(h2, wb_ref[...], preferred_element_type=jnp.float32)


def _tc3(pa, pb, tp, dinv, b2, Wb0):
    return pl.pallas_call(
        _tc3_body,
        out_shape=[jax.ShapeDtypeStruct((N, F), jnp.float32),
                   jax.ShapeDtypeStruct((N, F), jnp.float32)],
    )(pa, pb, tp, dinv, b2, Wb0)


def kernel(x, edge_index, W0, b0, W1, b1, W2, b2, Wb, bb):
    src = edge_index[0]
    dst = edge_index[1]
    src2 = src.reshape(E // CHP, CHP)
    dst2 = dst.reshape(E // CHP, CHP)

    degp = _sc_deg(dst)
    t1p, dinv = _tc1(x, W0, b0, W1, degp.T)

    zeros = jnp.zeros((N, F), jnp.float32)
    p1 = _sc_prop(t1p, src2, dst2, zeros)
    t2p = _tc2(p1[0], p1[1], t1p, dinv, b1, W2)
    p2 = _sc_prop(t2p, src2, dst2, zeros)
    h2, u = _tc3(p2[0], p2[1], t2p, dinv, b2, Wb[0])

    bb16 = jnp.full((16,), bb[0], jnp.float32)
    return _sc_score(u, h2, src2, dst2, bb16)


# speedup vs baseline: 3.0661x; 3.0661x over previous
import functools
import jax
import jax.numpy as jnp
from jax import lax
from jax.experimental import pallas as pl
from jax.experimental.pallas import tpu as pltpu, tpu_sc as plsc

N = 10000
D = 128
F = 64
E = 320000
NC = 2
NS = 16
NW = NC * NS
EPW = E // NW
CH = 128
NFULL = EPW // CH
TAIL = EPW - NFULL * CH
CHP = 80
NCHP = EPW // CHP

_mesh = functools.partial(
    plsc.VectorSubcoreMesh, core_axis_name="c", subcore_axis_name="s")


def _wid():
    return lax.axis_index("c") * NS + lax.axis_index("s")


@functools.partial(
    pl.kernel,
    out_type=jax.ShapeDtypeStruct((NW, N), jnp.float32),
    mesh=_mesh(),
    compiler_params=pltpu.CompilerParams(needs_layout_passes=False, use_tc_tiling_on_sc=False),
    scratch_types=[
        pltpu.VMEM((EPW,), jnp.int32),
        pltpu.VMEM((N,), jnp.float32),
        pltpu.SemaphoreType.DMA,
    ],
)
def _sc_deg(dst_hbm, out_hbm, dstv, degv, sem):
    wid = _wid()
    base = wid * EPW

    zero16 = jnp.zeros((16,), jnp.float32)

    def zbody(i, _):
        degv[pl.ds(i * 16, 16)] = zero16
        return 0
    lax.fori_loop(0, N // 16, zbody, 0)

    pltpu.sync_copy(dst_hbm.at[pl.ds(base, EPW)], dstv)

    one16 = jnp.ones((16,), jnp.float32)

    def body(i, _):
        idx = dstv[pl.ds(i * 16, 16)]
        plsc.addupdate_scatter(degv, [idx], one16)
        return 0
    lax.fori_loop(0, EPW // 16, body, 0)

    pltpu.sync_copy(degv, out_hbm.at[wid])


@functools.partial(
    pl.kernel,
    out_type=jax.ShapeDtypeStruct((NC, N, F), jnp.float32),
    mesh=_mesh(),
    compiler_params=pltpu.CompilerParams(needs_layout_passes=False, use_tc_tiling_on_sc=False),
    scratch_types=[
        pltpu.VMEM((NCHP, CHP), jnp.int32),
        pltpu.VMEM((NCHP, CHP), jnp.int32),
        pltpu.VMEM((CHP, F), jnp.float32),
        pltpu.VMEM((CHP, F), jnp.float32),
        pltpu.VMEM_SHARED((N, F), jnp.float32),
        pltpu.VMEM_SHARED((N, F), jnp.float32),
        pltpu.SemaphoreType.DMA,
        pltpu.SemaphoreType.DMA,
    ],
)
def _sc_prop(tp_hbm, src2_hbm, dst2_hbm, zeros_hbm, out_hbm,
             srcv, dstv, rowsa, rowsb, acc, tsh, sema, semb):
    c = lax.axis_index("c")
    s = lax.axis_index("s")
    row0 = _wid() * NCHP

    @pl.when(s < 10)
    def _():
        pltpu.sync_copy(zeros_hbm.at[pl.ds(s * 1000, 1000)],
                        acc.at[pl.ds(s * 1000, 1000)])
        pltpu.sync_copy(tp_hbm.at[pl.ds(s * 1000, 1000)],
                        tsh.at[pl.ds(s * 1000, 1000)])
    pltpu.sync_copy(src2_hbm.at[pl.ds(row0, NCHP)], srcv)
    pltpu.sync_copy(dst2_hbm.at[pl.ds(row0, NCHP)], dstv)
    plsc.subcore_barrier()

    def start(j, buf, sem):
        pltpu.async_copy(tsh.at[srcv.at[j]], buf, sem)

    def wait(buf, sem):
        pltpu.make_async_copy(tsh.at[srcv.at[0]], buf, sem).wait()

    def scat(j, buf):
        pltpu.sync_copy(buf, acc.at[dstv.at[j]], add=True)

    start(0, rowsa, sema)

    def pair(jj, _):
        j = 2 * jj
        start(j + 1, rowsb, semb)
        wait(rowsa, sema)
        scat(j, rowsa)

        @pl.when(jj < NCHP // 2 - 1)
        def _():
            start(j + 2, rowsa, sema)
        wait(rowsb, semb)
        scat(j + 1, rowsb)
        return 0
    lax.fori_loop(0, NCHP // 2, pair, 0)

    start(NCHP - 1, rowsa, sema)
    wait(rowsa, sema)
    scat(NCHP - 1, rowsa)

    plsc.subcore_barrier()

    @pl.when(s < 10)
    def _():
        pltpu.sync_copy(acc.at[pl.ds(s * 1000, 1000)],
                        out_hbm.at[c, pl.ds(s * 1000, 1000)])


@functools.partial(
    pl.kernel,
    out_type=jax.ShapeDtypeStruct((E,), jnp.float32),
    mesh=_mesh(),
    compiler_params=pltpu.CompilerParams(needs_layout_passes=False, use_tc_tiling_on_sc=False),
    scratch_types=[
        pltpu.VMEM((NCHP, CHP), jnp.int32),
        pltpu.VMEM((NCHP, CHP), jnp.int32),
        pltpu.VMEM((CHP, F), jnp.float32),
        pltpu.VMEM((CHP, F), jnp.float32),
        pltpu.VMEM((CHP, F), jnp.float32),
        pltpu.VMEM((CHP, F), jnp.float32),
        pltpu.VMEM((EPW,), jnp.float32),
        pltpu.VMEM((16, 16), jnp.float32),
        pltpu.VMEM((16,), jnp.float32),
        pltpu.VMEM_SHARED((N, F), jnp.float32),
        pltpu.VMEM_SHARED((N, F), jnp.float32),
        pltpu.SemaphoreType.DMA,
        pltpu.SemaphoreType.DMA,
    ],
)
def _sc_score(u_hbm, h_hbm, src2_hbm, dst2_hbm, bb_hbm, out_hbm,
              srcv, dstv, ua, ha, ub, hb, scores, ptile, bbv, ush, hsh,
              sema, semb):
    s = lax.axis_index("s")
    row0 = _wid() * NCHP
    pltpu.sync_copy(bb_hbm, bbv)
    bbvec = bbv[...]
    pltpu.sync_copy(src2_hbm.at[pl.ds(row0, NCHP)], srcv)
    pltpu.sync_copy(dst2_hbm.at[pl.ds(row0, NCHP)], dstv)

    @pl.when(s < 10)
    def _():
        pltpu.sync_copy(u_hbm.at[pl.ds(s * 1000, 1000)],
                        ush.at[pl.ds(s * 1000, 1000)])
        pltpu.sync_copy(h_hbm.at[pl.ds(s * 1000, 1000)],
                        hsh.at[pl.ds(s * 1000, 1000)])
    plsc.subcore_barrier()

    iota16 = lax.iota(jnp.int32, 16)

    def start(j, ubuf, hbuf, sem):
        pltpu.async_copy(ush.at[srcv.at[j]], ubuf, sem)
        pltpu.async_copy(hsh.at[dstv.at[j]], hbuf, sem)

    def wait(ubuf, hbuf, sem):
        pltpu.make_async_copy(ush.at[srcv.at[0]], ubuf, sem).wait()
        pltpu.make_async_copy(hsh.at[dstv.at[0]], hbuf, sem).wait()

    def compute(j, ubuf, hbuf):
        def group(g, _):
            for e in range(16):
                erow = ubuf.at[g * 16 + e]
                hrow = hbuf.at[g * 16 + e]
                p0 = erow[pl.ds(0, 16)] * hrow[pl.ds(0, 16)]
                p1 = erow[pl.ds(16, 16)] * hrow[pl.ds(16, 16)]
                p2 = erow[pl.ds(32, 16)] * hrow[pl.ds(32, 16)]
                p3 = erow[pl.ds(48, 16)] * hrow[pl.ds(48, 16)]
                ptile[e] = (p0 + p1) + (p2 + p3)
            acc = bbvec
            for c in range(16):
                colc = jnp.full((16,), c, jnp.int32)
                acc = acc + plsc.load_gather(ptile, [iota16, colc])
            scores[pl.ds(j * CHP + g * 16, 16)] = acc
            return 0
        lax.fori_loop(0, CHP // 16, group, 0)

    start(0, ua, ha, sema)

    def pair(jj, _):
        j = 2 * jj
        start(j + 1, ub, hb, semb)
        wait(ua, ha, sema)
        compute(j, ua, ha)

        @pl.when(jj < NCHP // 2 - 1)
        def _():
            start(j + 2, ua, ha, sema)
        wait(ub, hb, semb)
        compute(j + 1, ub, hb)
        return 0
    lax.fori_loop(0, NCHP // 2, pair, 0)

    start(NCHP - 1, ua, ha, sema)
    wait(ua, ha, sema)
    compute(NCHP - 1, ua, ha)

    pltpu.sync_copy(scores, out_hbm.at[pl.ds(_wid() * EPW, EPW)])


def _tc1_body(x_ref, w0_ref, b0_ref, w1_ref, deg_ref, t1p_ref, dinv_ref):
    h0 = jnp.dot(x_ref[...], w0_ref[...],
                 preferred_element_type=jnp.float32) + b0_ref[...]
    t1 = jnp.dot(h0, w1_ref[...], preferred_element_type=jnp.float32)
    deg = jnp.sum(deg_ref[...], axis=1, keepdims=True) + 1.0
    dinv = lax.rsqrt(deg)
    dinv_ref[...] = dinv
    t1p_ref[...] = t1 * dinv


def _tc1(x, W0, b0, W1, deg3):
    return pl.pallas_call(
        _tc1_body,
        out_shape=[jax.ShapeDtypeStruct((N, F), jnp.float32),
                   jax.ShapeDtypeStruct((N, 1), jnp.float32)],
    )(x, W0, b0, W1, deg3)


def _tc2_body(pa_ref, pb_ref, tp_ref, dinv_ref, b1_ref, w2_ref, out_ref):
    dinv = dinv_ref[...]
    ssum = pa_ref[...] + pb_ref[...] + tp_ref[...]
    h1 = jnp.maximum(ssum * dinv + b1_ref[...], 0.0)
    t2 = jnp.dot(h1, w2_ref[...], preferred_element_type=jnp.float32)
    out_ref[...] = t2 * dinv


def _tc2(pa, pb, tp, dinv, b1, W2):
    return pl.pallas_call(
        _tc2_body,
        out_shape=jax.ShapeDtypeStruct((N, F), jnp.float32),
    )(pa, pb, tp, dinv, b1, W2)


def _tc3_body(pa_ref, pb_ref, tp_ref, dinv_ref, b2_ref, wb_ref,
              h2_ref, u_ref):
    ssum = pa_ref[...] + pb_ref[...] + tp_ref[...]
    h2 = ssum * dinv_ref[...] + b2_ref[...]
    h2_ref[...] = h2
    u_ref[...] = jnp.dot(h2, wb_ref[...], preferred_element_type=jnp.float32)


def _tc3(pa, pb, tp, dinv, b2, Wb0):
    return pl.pallas_call(
        _tc3_body,
        out_shape=[jax.ShapeDtypeStruct((N, F), jnp.float32),
                   jax.ShapeDtypeStruct((N, F), jnp.float32)],
    )(pa, pb, tp, dinv, b2, Wb0)


def kernel(x, edge_index, W0, b0, W1, b1, W2, b2, Wb, bb):
    src = edge_index[0]
    dst = edge_index[1]
    src2 = src.reshape(E // CHP, CHP)
    dst2 = dst.reshape(E // CHP, CHP)

    degp = _sc_deg(dst)
    t1p, dinv = _tc1(x, W0, b0, W1, degp.T)

    zeros = jnp.zeros((N, F), jnp.float32)
    p1 = _sc_prop(t1p, src2, dst2, zeros)
    t2p = _tc2(p1[0], p1[1], t1p, dinv, b1, W2)
    p2 = _sc_prop(t2p, src2, dst2, zeros)
    h2, u = _tc3(p2[0], p2[1], t2p, dinv, b2, Wb[0])

    bb16 = jnp.full((16,), bb[0], jnp.float32)
    return _sc_score(u, h2, src2, dst2, bb16)
